# Initial kernel scaffold; baseline (speedup 1.0000x reference)
import jax
import jax.numpy as jnp
from jax.experimental import pallas as pl


def kernel(inp, out_grid_displacement, in_grid_displacement, initial_mesh, W_lift, b_lift, W_k1, b_k1, W_k2, b_k2, W_proj, b_proj, ln_g, ln_b, W_branch, b_branch, W_t1, b_t1, W_t2, b_t2, bias):
    raise NotImplementedError("write your pallas kernel here")


# speedup vs baseline: 4.9280x; 4.9280x over previous
import functools

import jax
import jax.numpy as jnp
import numpy as np
from jax import lax
from jax.experimental import pallas as pl
from jax.experimental.pallas import tpu as pltpu
from jax.experimental.pallas import tpu_sc as plsc

N = 10000
K = 10
H = 64
OUT_DIM = 3

NPAD = 10240
NB = 512
NSTEPS = NPAD // NB
CPAD = 10112
QB = 128
E_PAD = K * NPAD

SC_NC = 2
SC_NS = 16
SC_NW = SC_NC * SC_NS
E_PER_W = E_PAD // SC_NW
SC_CHUNK = 128
SC_NCHUNK = E_PER_W // SC_CHUNK

FREQS = ((1.0 / 10000.0) ** (np.arange(4, dtype=np.float32) /
                             np.float32(4.0))).astype(np.float32)


def _pe8(col):
    ang = jnp.concatenate([col * float(FREQS[i]) for i in range(4)], axis=1)
    return jnp.concatenate([jnp.cos(ang), jnp.sin(ang)], axis=1)


def _prep_body(im_ref, ind_ref, outd_ref, inp0_ref,
               wl_ref, bl_ref, wt1_ref, bt1_ref, wt2_ref, bt2_ref,
               ing_ref, table_ref, tout_ref):
    im = im_ref[...]
    ing = im + ind_ref[...]
    outg = im + outd_ref[...]
    ing_ref[...] = ing

    in_pe = jnp.concatenate([_pe8(ing[:, 0:1]), _pe8(ing[:, 1:2])], axis=1)
    in_data = jnp.concatenate([inp0_ref[...], in_pe], axis=1)
    f = jnp.dot(in_data, wl_ref[...],
                preferred_element_type=jnp.float32) + bl_ref[...]
    table_ref[...] = jnp.concatenate(
        [f, ing, jnp.zeros((NB, 62), jnp.float32)], axis=1)

    out_pe = jnp.concatenate([_pe8(outg[:, 0:1]), _pe8(outg[:, 1:2])], axis=1)
    grid_pe = jnp.concatenate([outg, out_pe], axis=1)
    t1 = jnp.maximum(jnp.dot(grid_pe, wt1_ref[...],
                             preferred_element_type=jnp.float32) + bt1_ref[...], 0.0)
    t2 = jnp.maximum(jnp.dot(t1, wt2_ref[...],
                             preferred_element_type=jnp.float32) + bt2_ref[...], 0.0)
    tout_ref[...] = t2


def _knn_body(qx_ref, qy_ref, px_ref, py_ref, out_ref):
    dx = qx_ref[...] - px_ref[...]
    dy = qy_ref[...] - py_ref[...]
    d = dx * dx + dy * dy
    cols = lax.broadcasted_iota(jnp.int32, (QB, CPAD), 1)
    idxs = []
    for _ in range(K):
        m = jnp.min(d, axis=1, keepdims=True)
        sel = jnp.where(d == m, cols, jnp.int32(2 ** 30))
        idx = jnp.min(sel, axis=1, keepdims=True)
        idxs.append(idx)
        d = jnp.where(cols == idx, jnp.float32(np.inf), d)
    idxs.append(jnp.zeros((QB, 16 - K), jnp.int32))
    out_ref[...] = jnp.concatenate(idxs, axis=1)


def _edge_body(g_ref, ing_ref, wk1_ref, bk1_ref, wk2_ref, bk2_ref,
               wp_ref, bp_ref, lng_ref, lnb_ref, bn_ref):
    y = ing_ref[...]
    acc = jnp.zeros((NB, H), jnp.float32)
    for kk in range(K):
        gk = g_ref[kk]
        fnb = gk[:, 0:H]
        xnb = gk[:, H:H + 2]
        edge = jnp.concatenate([y, xnb], axis=1)
        h = jax.nn.gelu(jnp.dot(edge, wk1_ref[...],
                                preferred_element_type=jnp.float32) + bk1_ref[...])
        kern = jnp.dot(h, wk2_ref[...],
                       preferred_element_type=jnp.float32) + bk2_ref[...]
        acc = acc + kern * fnb
    agg = acc / jnp.float32(K)
    gout = jnp.dot(agg, wp_ref[...],
                   preferred_element_type=jnp.float32) + bp_ref[...]
    mu = jnp.mean(gout, axis=1, keepdims=True)
    xc = gout - mu
    var = jnp.mean(xc * xc, axis=1, keepdims=True)
    bn_ref[...] = xc / jnp.sqrt(var + 1e-5) * lng_ref[...] + lnb_ref[...]


def _branch_body(x_ref, w_ref, b_ref, out_ref):
    @pl.when(pl.program_id(0) == 0)
    def _():
        out_ref[...] = b_ref[...]
    out_ref[...] += jnp.dot(x_ref[...], w_ref[...],
                            preferred_element_type=jnp.float32)


def _final_body(t_ref, bout_ref, bias_ref, out_ref):
    t = t_ref[...]
    bs = bout_ref[...] * jnp.float32(0.125)
    cols = []
    for c in range(OUT_DIM):
        s = jnp.sum(t[:, H * c:H * (c + 1)] * bs, axis=1, keepdims=True)
        cols.append(s + bias_ref[0:1, c:c + 1])
    cols.append(jnp.zeros((NB, 1), jnp.float32))
    out_ref[...] = jnp.concatenate(cols, axis=1)


def _sc_gather_body(table_hbm, idx_hbm, out_hbm, idx_v, rows_v, sem):
    wid = lax.axis_index("s") * SC_NC + lax.axis_index("c")

    def body(j, carry):
        base = pl.multiple_of(wid * E_PER_W + j * SC_CHUNK, 8)
        pltpu.sync_copy(idx_hbm.at[pl.ds(base, SC_CHUNK)], idx_v)
        pltpu.async_copy(table_hbm.at[idx_v], rows_v, sem).wait()
        pltpu.sync_copy(rows_v, out_hbm.at[pl.ds(base, SC_CHUNK)])
        return carry

    lax.fori_loop(0, SC_NCHUNK, body, 0)


def _sc_gather(table, idx_flat):
    fn = pl.kernel(
        _sc_gather_body,
        mesh=plsc.VectorSubcoreMesh(core_axis_name="c", subcore_axis_name="s",
                                    num_cores=SC_NC, num_subcores=SC_NS),
        out_type=jax.ShapeDtypeStruct((E_PAD, 128), jnp.float32),
        scratch_types=[
            pltpu.VMEM((SC_CHUNK,), jnp.int32),
            pltpu.VMEM((SC_CHUNK, 128), jnp.float32),
            pltpu.SemaphoreType.DMA,
        ],
    )
    return fn(table, idx_flat)


def _pad_rows(a, rows):
    return jnp.concatenate(
        [a, jnp.zeros((rows - a.shape[0],) + a.shape[1:], a.dtype)], axis=0)


def kernel(inp, out_grid_displacement, in_grid_displacement, initial_mesh,
           W_lift, b_lift, W_k1, b_k1, W_k2, b_k2, W_proj, b_proj,
           ln_g, ln_b, W_branch, b_branch, W_t1, b_t1, W_t2, b_t2, bias):
    f32 = jnp.float32
    im = _pad_rows(initial_mesh, NPAD)
    ind = _pad_rows(in_grid_displacement, NPAD)
    outd = _pad_rows(out_grid_displacement, NPAD)
    inp0 = _pad_rows(inp[0], NPAD)

    row = lambda v: v.reshape(1, -1)

    ing_p, table, tout = pl.pallas_call(
        _prep_body,
        grid=(NSTEPS,),
        in_specs=[
            pl.BlockSpec((NB, 2), lambda j: (j, 0)),
            pl.BlockSpec((NB, 2), lambda j: (j, 0)),
            pl.BlockSpec((NB, 2), lambda j: (j, 0)),
            pl.BlockSpec((NB, 3), lambda j: (j, 0)),
            pl.BlockSpec((19, H), lambda j: (0, 0)),
            pl.BlockSpec((1, H), lambda j: (0, 0)),
            pl.BlockSpec((18, H), lambda j: (0, 0)),
            pl.BlockSpec((1, H), lambda j: (0, 0)),
            pl.BlockSpec((H, H * OUT_DIM), lambda j: (0, 0)),
            pl.BlockSpec((1, H * OUT_DIM), lambda j: (0, 0)),
        ],
        out_specs=[
            pl.BlockSpec((NB, 2), lambda j: (j, 0)),
            pl.BlockSpec((NB, 128), lambda j: (j, 0)),
            pl.BlockSpec((NB, H * OUT_DIM), lambda j: (j, 0)),
        ],
        out_shape=[
            jax.ShapeDtypeStruct((NPAD, 2), f32),
            jax.ShapeDtypeStruct((NPAD, 128), f32),
            jax.ShapeDtypeStruct((NPAD, H * OUT_DIM), f32),
        ],
    )(im, ind, outd, inp0, W_lift, row(b_lift), W_t1, row(b_t1),
      W_t2, row(b_t2))

    qx = ing_p[:, 0:1]
    qy = ing_p[:, 1:2]
    px = jnp.full((1, CPAD), 1e9, f32).at[0, :N].set(ing_p[:N, 0])
    py = jnp.full((1, CPAD), 1e9, f32).at[0, :N].set(ing_p[:N, 1])
    nbrs16 = pl.pallas_call(
        _knn_body,
        grid=(NPAD // QB,),
        in_specs=[
            pl.BlockSpec((QB, 1), lambda j: (j, 0)),
            pl.BlockSpec((QB, 1), lambda j: (j, 0)),
            pl.BlockSpec((1, CPAD), lambda j: (0, 0)),
            pl.BlockSpec((1, CPAD), lambda j: (0, 0)),
        ],
        out_specs=pl.BlockSpec((QB, 16), lambda j: (j, 0)),
        out_shape=jax.ShapeDtypeStruct((NPAD, 16), jnp.int32),
    )(qx, qy, px, py)

    idx_flat = jnp.zeros((K, NPAD), jnp.int32)
    idx_flat = idx_flat.at[:, :N].set(nbrs16[:N, :K].T).reshape(E_PAD)

    g = _sc_gather(table, idx_flat)
    g3 = g.reshape(K, NPAD, 128)

    bnorm = pl.pallas_call(
        _edge_body,
        grid=(NSTEPS,),
        in_specs=[
            pl.BlockSpec((K, NB, 128), lambda j: (0, j, 0)),
            pl.BlockSpec((NB, 2), lambda j: (j, 0)),
            pl.BlockSpec((4, H), lambda j: (0, 0)),
            pl.BlockSpec((1, H), lambda j: (0, 0)),
            pl.BlockSpec((H, H), lambda j: (0, 0)),
            pl.BlockSpec((1, H), lambda j: (0, 0)),
            pl.BlockSpec((H, H), lambda j: (0, 0)),
            pl.BlockSpec((1, H), lambda j: (0, 0)),
            pl.BlockSpec((1, H), lambda j: (0, 0)),
            pl.BlockSpec((1, H), lambda j: (0, 0)),
        ],
        out_specs=pl.BlockSpec((NB, H), lambda j: (j, 0)),
        out_shape=jax.ShapeDtypeStruct((NPAD, H), f32),
    )(g3, ing_p, W_k1, row(b_k1), W_k2, row(b_k2), W_proj, row(b_proj),
      row(ln_g), row(ln_b))

    x = bnorm[:N].reshape(1, N * H)
    RB = 12800
    bout = pl.pallas_call(
        _branch_body,
        grid=(N * H // RB,),
        in_specs=[
            pl.BlockSpec((1, RB), lambda j: (0, j)),
            pl.BlockSpec((RB, H), lambda j: (j, 0)),
            pl.BlockSpec((1, H), lambda j: (0, 0)),
        ],
        out_specs=pl.BlockSpec((1, H), lambda j: (0, 0)),
        out_shape=jax.ShapeDtypeStruct((1, H), f32),
    )(x, W_branch, row(b_branch))

    bias128 = jnp.zeros((1, 128), f32).at[0, :OUT_DIM].set(bias)
    out_p = pl.pallas_call(
        _final_body,
        grid=(NSTEPS,),
        in_specs=[
            pl.BlockSpec((NB, H * OUT_DIM), lambda j: (j, 0)),
            pl.BlockSpec((1, H), lambda j: (0, 0)),
            pl.BlockSpec((1, 128), lambda j: (0, 0)),
        ],
        out_specs=pl.BlockSpec((NB, 4), lambda j: (j, 0)),
        out_shape=jax.ShapeDtypeStruct((NPAD, 4), f32),
    )(tout, bout, bias128)

    return out_p[:N, :OUT_DIM][None]


# speedup vs baseline: 7.0014x; 1.4207x over previous
import functools

import jax
import jax.numpy as jnp
import numpy as np
from jax import lax
from jax.experimental import pallas as pl
from jax.experimental.pallas import tpu as pltpu
from jax.experimental.pallas import tpu_sc as plsc

N = 10000
K = 10
H = 64
OUT_DIM = 3

NPAD = 10240
NB = 512
NSTEPS = NPAD // NB
CPAD = 10112
QB = 128
E_PAD = K * NPAD

SC_NC = 2
SC_NS = 16
SC_NW = SC_NC * SC_NS
E_PER_W = E_PAD // SC_NW
SC_CHUNK = 128
SC_NCHUNK = E_PER_W // SC_CHUNK

FREQS = ((1.0 / 10000.0) ** (np.arange(4, dtype=np.float32) /
                             np.float32(4.0))).astype(np.float32)


def _pe8(col):
    ang = jnp.concatenate([col * float(FREQS[i]) for i in range(4)], axis=1)
    return jnp.concatenate([jnp.cos(ang), jnp.sin(ang)], axis=1)


def _prep_body(im_ref, ind_ref, outd_ref, inp0_ref,
               wl_ref, bl_ref, wt1_ref, bt1_ref, wt2_ref, bt2_ref,
               ing_ref, table_ref, tout_ref):
    im = im_ref[...]
    ing = im + ind_ref[...]
    outg = im + outd_ref[...]
    ing_ref[...] = ing

    in_pe = jnp.concatenate([_pe8(ing[:, 0:1]), _pe8(ing[:, 1:2])], axis=1)
    in_data = jnp.concatenate([inp0_ref[...], in_pe], axis=1)
    f = jnp.dot(in_data, wl_ref[...],
                preferred_element_type=jnp.float32) + bl_ref[...]
    table_ref[...] = jnp.concatenate(
        [f, ing, jnp.zeros((NB, 62), jnp.float32)], axis=1)

    out_pe = jnp.concatenate([_pe8(outg[:, 0:1]), _pe8(outg[:, 1:2])], axis=1)
    grid_pe = jnp.concatenate([outg, out_pe], axis=1)
    t1 = jnp.maximum(jnp.dot(grid_pe, wt1_ref[...],
                             preferred_element_type=jnp.float32) + bt1_ref[...], 0.0)
    t2 = jnp.maximum(jnp.dot(t1, wt2_ref[...],
                             preferred_element_type=jnp.float32) + bt2_ref[...], 0.0)
    tout_ref[...] = t2


NG = CPAD // 8


def _knn_body(qx_ref, qy_ref, px_ref, py_ref, pxr_ref, pyr_ref, out_ref):
    BIGI = jnp.int32(2 ** 30)
    INF = jnp.float32(np.inf)
    qx = qx_ref[...]
    qy = qy_ref[...]
    dx = qx - px_ref[...]
    dy = qy - py_ref[...]
    d = dx * dx + dy * dy
    m_arr = d[:, 0:NG]
    for m in range(1, 8):
        m_arr = jnp.minimum(m_arr, d[:, m * NG:(m + 1) * NG])
    gcols = lax.broadcasted_iota(jnp.int32, (QB, NG), 1)
    gids = []
    for _ in range(K):
        mn = jnp.min(m_arr, axis=1, keepdims=True)
        gid = jnp.min(jnp.where(m_arr == mn, gcols, BIGI),
                      axis=1, keepdims=True)
        gids.append(gid)
        m_arr = jnp.where(gcols == gid, INF, m_arr)
    offs = lax.broadcasted_iota(jnp.int32, (QB, 8), 1)
    cpx, cpy, ccol = [], [], []
    for t in range(K):
        mask = (gcols == gids[t]).astype(jnp.float32)
        cpx.append(jnp.dot(mask, pxr_ref[...],
                           preferred_element_type=jnp.float32))
        cpy.append(jnp.dot(mask, pyr_ref[...],
                           preferred_element_type=jnp.float32))
        ccol.append(gids[t] + offs * NG)
    cand_px = jnp.concatenate(cpx, axis=1)
    cand_py = jnp.concatenate(cpy, axis=1)
    ccols = jnp.concatenate(ccol, axis=1)
    ddx = qx - cand_px
    ddy = qy - cand_py
    dc = ddx * ddx + ddy * ddy
    idxs = []
    for _ in range(K):
        mn = jnp.min(dc, axis=1, keepdims=True)
        idx = jnp.min(jnp.where(dc == mn, ccols, BIGI),
                      axis=1, keepdims=True)
        idxs.append(idx)
        dc = jnp.where(ccols == idx, INF, dc)
    idxs.append(jnp.zeros((QB, 16 - K), jnp.int32))
    out_ref[...] = jnp.concatenate(idxs, axis=1)


def _edge_body(g_ref, ing_ref, wk1_ref, bk1_ref, wk2_ref, bk2_ref,
               wp_ref, bp_ref, lng_ref, lnb_ref, bn_ref):
    y = ing_ref[...]
    acc = jnp.zeros((NB, H), jnp.float32)
    for kk in range(K):
        gk = g_ref[kk]
        fnb = gk[:, 0:H]
        xnb = gk[:, H:H + 2]
        edge = jnp.concatenate([y, xnb], axis=1)
        h = jax.nn.gelu(jnp.dot(edge, wk1_ref[...],
                                preferred_element_type=jnp.float32) + bk1_ref[...])
        kern = jnp.dot(h, wk2_ref[...],
                       preferred_element_type=jnp.float32) + bk2_ref[...]
        acc = acc + kern * fnb
    agg = acc / jnp.float32(K)
    gout = jnp.dot(agg, wp_ref[...],
                   preferred_element_type=jnp.float32) + bp_ref[...]
    mu = jnp.mean(gout, axis=1, keepdims=True)
    xc = gout - mu
    var = jnp.mean(xc * xc, axis=1, keepdims=True)
    bn_ref[...] = xc / jnp.sqrt(var + 1e-5) * lng_ref[...] + lnb_ref[...]


def _branch_body(x_ref, w_ref, b_ref, out_ref):
    @pl.when(pl.program_id(0) == 0)
    def _():
        out_ref[...] = b_ref[...]
    out_ref[...] += jnp.dot(x_ref[...], w_ref[...],
                            preferred_element_type=jnp.float32)


def _final_body(t_ref, bout_ref, bias_ref, out_ref):
    t = t_ref[...]
    bs = bout_ref[...] * jnp.float32(0.125)
    cols = []
    for c in range(OUT_DIM):
        s = jnp.sum(t[:, H * c:H * (c + 1)] * bs, axis=1, keepdims=True)
        cols.append(s + bias_ref[0:1, c:c + 1])
    cols.append(jnp.zeros((NB, 1), jnp.float32))
    out_ref[...] = jnp.concatenate(cols, axis=1)


def _sc_gather_body(table_hbm, idx_hbm, out_hbm, idx_v, rows_v, sem):
    wid = lax.axis_index("s") * SC_NC + lax.axis_index("c")

    def body(j, carry):
        base = pl.multiple_of(wid * E_PER_W + j * SC_CHUNK, 8)
        pltpu.sync_copy(idx_hbm.at[pl.ds(base, SC_CHUNK)], idx_v)
        pltpu.async_copy(table_hbm.at[idx_v], rows_v, sem).wait()
        pltpu.sync_copy(rows_v, out_hbm.at[pl.ds(base, SC_CHUNK)])
        return carry

    lax.fori_loop(0, SC_NCHUNK, body, 0)


def _sc_gather(table, idx_flat):
    fn = pl.kernel(
        _sc_gather_body,
        mesh=plsc.VectorSubcoreMesh(core_axis_name="c", subcore_axis_name="s",
                                    num_cores=SC_NC, num_subcores=SC_NS),
        out_type=jax.ShapeDtypeStruct((E_PAD, 128), jnp.float32),
        scratch_types=[
            pltpu.VMEM((SC_CHUNK,), jnp.int32),
            pltpu.VMEM((SC_CHUNK, 128), jnp.float32),
            pltpu.SemaphoreType.DMA,
        ],
    )
    return fn(table, idx_flat)


def _pad_rows(a, rows):
    return jnp.concatenate(
        [a, jnp.zeros((rows - a.shape[0],) + a.shape[1:], a.dtype)], axis=0)


def kernel(inp, out_grid_displacement, in_grid_displacement, initial_mesh,
           W_lift, b_lift, W_k1, b_k1, W_k2, b_k2, W_proj, b_proj,
           ln_g, ln_b, W_branch, b_branch, W_t1, b_t1, W_t2, b_t2, bias):
    f32 = jnp.float32
    im = _pad_rows(initial_mesh, NPAD)
    ind = _pad_rows(in_grid_displacement, NPAD)
    outd = _pad_rows(out_grid_displacement, NPAD)
    inp0 = _pad_rows(inp[0], NPAD)

    row = lambda v: v.reshape(1, -1)

    ing_p, table, tout = pl.pallas_call(
        _prep_body,
        grid=(NSTEPS,),
        in_specs=[
            pl.BlockSpec((NB, 2), lambda j: (j, 0)),
            pl.BlockSpec((NB, 2), lambda j: (j, 0)),
            pl.BlockSpec((NB, 2), lambda j: (j, 0)),
            pl.BlockSpec((NB, 3), lambda j: (j, 0)),
            pl.BlockSpec((19, H), lambda j: (0, 0)),
            pl.BlockSpec((1, H), lambda j: (0, 0)),
            pl.BlockSpec((18, H), lambda j: (0, 0)),
            pl.BlockSpec((1, H), lambda j: (0, 0)),
            pl.BlockSpec((H, H * OUT_DIM), lambda j: (0, 0)),
            pl.BlockSpec((1, H * OUT_DIM), lambda j: (0, 0)),
        ],
        out_specs=[
            pl.BlockSpec((NB, 2), lambda j: (j, 0)),
            pl.BlockSpec((NB, 128), lambda j: (j, 0)),
            pl.BlockSpec((NB, H * OUT_DIM), lambda j: (j, 0)),
        ],
        out_shape=[
            jax.ShapeDtypeStruct((NPAD, 2), f32),
            jax.ShapeDtypeStruct((NPAD, 128), f32),
            jax.ShapeDtypeStruct((NPAD, H * OUT_DIM), f32),
        ],
    )(im, ind, outd, inp0, W_lift, row(b_lift), W_t1, row(b_t1),
      W_t2, row(b_t2))

    qx = ing_p[:, 0:1]
    qy = ing_p[:, 1:2]
    px = jnp.full((1, CPAD), 1e9, f32).at[0, :N].set(ing_p[:N, 0])
    py = jnp.full((1, CPAD), 1e9, f32).at[0, :N].set(ing_p[:N, 1])
    pxr = px.reshape(8, NG).T
    pyr = py.reshape(8, NG).T
    nbrs16 = pl.pallas_call(
        _knn_body,
        grid=(NPAD // QB,),
        in_specs=[
            pl.BlockSpec((QB, 1), lambda j: (j, 0)),
            pl.BlockSpec((QB, 1), lambda j: (j, 0)),
            pl.BlockSpec((1, CPAD), lambda j: (0, 0)),
            pl.BlockSpec((1, CPAD), lambda j: (0, 0)),
            pl.BlockSpec((NG, 8), lambda j: (0, 0)),
            pl.BlockSpec((NG, 8), lambda j: (0, 0)),
        ],
        out_specs=pl.BlockSpec((QB, 16), lambda j: (j, 0)),
        out_shape=jax.ShapeDtypeStruct((NPAD, 16), jnp.int32),
    )(qx, qy, px, py, pxr, pyr)

    idx_flat = jnp.zeros((K, NPAD), jnp.int32)
    idx_flat = idx_flat.at[:, :N].set(nbrs16[:N, :K].T).reshape(E_PAD)

    g = _sc_gather(table, idx_flat)
    g3 = g.reshape(K, NPAD, 128)

    bnorm = pl.pallas_call(
        _edge_body,
        grid=(NSTEPS,),
        in_specs=[
            pl.BlockSpec((K, NB, 128), lambda j: (0, j, 0)),
            pl.BlockSpec((NB, 2), lambda j: (j, 0)),
            pl.BlockSpec((4, H), lambda j: (0, 0)),
            pl.BlockSpec((1, H), lambda j: (0, 0)),
            pl.BlockSpec((H, H), lambda j: (0, 0)),
            pl.BlockSpec((1, H), lambda j: (0, 0)),
            pl.BlockSpec((H, H), lambda j: (0, 0)),
            pl.BlockSpec((1, H), lambda j: (0, 0)),
            pl.BlockSpec((1, H), lambda j: (0, 0)),
            pl.BlockSpec((1, H), lambda j: (0, 0)),
        ],
        out_specs=pl.BlockSpec((NB, H), lambda j: (j, 0)),
        out_shape=jax.ShapeDtypeStruct((NPAD, H), f32),
    )(g3, ing_p, W_k1, row(b_k1), W_k2, row(b_k2), W_proj, row(b_proj),
      row(ln_g), row(ln_b))

    x = bnorm[:N].reshape(1, N * H)
    RB = 12800
    bout = pl.pallas_call(
        _branch_body,
        grid=(N * H // RB,),
        in_specs=[
            pl.BlockSpec((1, RB), lambda j: (0, j)),
            pl.BlockSpec((RB, H), lambda j: (j, 0)),
            pl.BlockSpec((1, H), lambda j: (0, 0)),
        ],
        out_specs=pl.BlockSpec((1, H), lambda j: (0, 0)),
        out_shape=jax.ShapeDtypeStruct((1, H), f32),
    )(x, W_branch, row(b_branch))

    bias128 = jnp.zeros((1, 128), f32).at[0, :OUT_DIM].set(bias)
    out_p = pl.pallas_call(
        _final_body,
        grid=(NSTEPS,),
        in_specs=[
            pl.BlockSpec((NB, H * OUT_DIM), lambda j: (j, 0)),
            pl.BlockSpec((1, H), lambda j: (0, 0)),
            pl.BlockSpec((1, 128), lambda j: (0, 0)),
        ],
        out_specs=pl.BlockSpec((NB, 4), lambda j: (j, 0)),
        out_shape=jax.ShapeDtypeStruct((NPAD, 4), f32),
    )(tout, bout, bias128)

    return out_p[:N, :OUT_DIM][None]


# speedup vs baseline: 7.0087x; 1.0010x over previous
import functools

import jax
import jax.numpy as jnp
import numpy as np
from jax import lax
from jax.experimental import pallas as pl
from jax.experimental.pallas import tpu as pltpu
from jax.experimental.pallas import tpu_sc as plsc

N = 10000
K = 10
H = 64
OUT_DIM = 3

NPAD = 10240
NB = 512
NSTEPS = NPAD // NB
CPAD = 10112
QB = 128
E_PAD = K * NPAD

SC_NC = 2
SC_NS = 16
SC_NW = SC_NC * SC_NS
E_PER_W = E_PAD // SC_NW
SC_CHUNK = 128
SC_NCHUNK = E_PER_W // SC_CHUNK

FREQS = ((1.0 / 10000.0) ** (np.arange(4, dtype=np.float32) /
                             np.float32(4.0))).astype(np.float32)


def _pe8(col):
    ang = jnp.concatenate([col * float(FREQS[i]) for i in range(4)], axis=1)
    return jnp.concatenate([jnp.cos(ang), jnp.sin(ang)], axis=1)


def _prep_body(im_ref, ind_ref, outd_ref, inp0_ref,
               wl_ref, bl_ref, wt1_ref, bt1_ref, wt2_ref, bt2_ref,
               ing_ref, table_ref, tout_ref):
    im = im_ref[...]
    ing = im + ind_ref[...]
    outg = im + outd_ref[...]
    ing_ref[...] = ing

    in_pe = jnp.concatenate([_pe8(ing[:, 0:1]), _pe8(ing[:, 1:2])], axis=1)
    in_data = jnp.concatenate([inp0_ref[...], in_pe], axis=1)
    f = jnp.dot(in_data, wl_ref[...],
                preferred_element_type=jnp.float32) + bl_ref[...]
    table_ref[...] = jnp.concatenate(
        [f, ing, jnp.zeros((NB, 62), jnp.float32)], axis=1)

    out_pe = jnp.concatenate([_pe8(outg[:, 0:1]), _pe8(outg[:, 1:2])], axis=1)
    grid_pe = jnp.concatenate([outg, out_pe], axis=1)
    t1 = jnp.maximum(jnp.dot(grid_pe, wt1_ref[...],
                             preferred_element_type=jnp.float32) + bt1_ref[...], 0.0)
    t2 = jnp.maximum(jnp.dot(t1, wt2_ref[...],
                             preferred_element_type=jnp.float32) + bt2_ref[...], 0.0)
    tout_ref[...] = t2


NG = CPAD // 8


def _knn_body(qx_ref, qy_ref, px_ref, py_ref, pxr_ref, pyr_ref, out_ref):
    BIGI = jnp.int32(2 ** 30)
    INF = jnp.float32(np.inf)
    qx = qx_ref[...]
    qy = qy_ref[...]
    dx = qx - px_ref[...]
    dy = qy - py_ref[...]
    d = dx * dx + dy * dy
    m_arr = d[:, 0:NG]
    for m in range(1, 8):
        m_arr = jnp.minimum(m_arr, d[:, m * NG:(m + 1) * NG])
    gcols = lax.broadcasted_iota(jnp.int32, (QB, NG), 1)
    gids = []
    for _ in range(K):
        mn = jnp.min(m_arr, axis=1, keepdims=True)
        gid = jnp.min(jnp.where(m_arr == mn, gcols, BIGI),
                      axis=1, keepdims=True)
        gids.append(gid)
        m_arr = jnp.where(gcols == gid, INF, m_arr)
    offs = lax.broadcasted_iota(jnp.int32, (QB, 8), 1)
    cpx, cpy, ccol = [], [], []
    for t in range(K):
        mask = (gcols == gids[t]).astype(jnp.float32)
        cpx.append(jnp.dot(mask, pxr_ref[...],
                           preferred_element_type=jnp.float32))
        cpy.append(jnp.dot(mask, pyr_ref[...],
                           preferred_element_type=jnp.float32))
        ccol.append(gids[t] + offs * NG)
    cand_px = jnp.concatenate(cpx, axis=1)
    cand_py = jnp.concatenate(cpy, axis=1)
    ccols = jnp.concatenate(ccol, axis=1)
    ddx = qx - cand_px
    ddy = qy - cand_py
    dc = ddx * ddx + ddy * ddy
    idxs = []
    for _ in range(K):
        mn = jnp.min(dc, axis=1, keepdims=True)
        idx = jnp.min(jnp.where(dc == mn, ccols, BIGI),
                      axis=1, keepdims=True)
        idxs.append(idx)
        dc = jnp.where(ccols == idx, INF, dc)
    idxs.append(jnp.zeros((QB, 16 - K), jnp.int32))
    out_ref[...] = jnp.concatenate(idxs, axis=1)


def _edge_body(g_ref, ing_ref, wk1_ref, bk1_ref, wk2_ref, bk2_ref,
               wp_ref, bp_ref, lng_ref, lnb_ref, bn_ref):
    y = ing_ref[...]
    acc = jnp.zeros((NB, H), jnp.float32)
    for kk in range(K):
        gk = g_ref[kk]
        fnb = gk[:, 0:H]
        xnb = gk[:, H:H + 2]
        edge = jnp.concatenate([y, xnb], axis=1)
        h = jax.nn.gelu(jnp.dot(edge, wk1_ref[...],
                                preferred_element_type=jnp.float32) + bk1_ref[...])
        kern = jnp.dot(h, wk2_ref[...],
                       preferred_element_type=jnp.float32) + bk2_ref[...]
        acc = acc + kern * fnb
    agg = acc / jnp.float32(K)
    gout = jnp.dot(agg, wp_ref[...],
                   preferred_element_type=jnp.float32) + bp_ref[...]
    mu = jnp.mean(gout, axis=1, keepdims=True)
    xc = gout - mu
    var = jnp.mean(xc * xc, axis=1, keepdims=True)
    bn_ref[...] = xc / jnp.sqrt(var + 1e-5) * lng_ref[...] + lnb_ref[...]


def _branch_body(x_ref, w_ref, b_ref, out_ref):
    @pl.when(pl.program_id(0) == 0)
    def _():
        out_ref[...] = b_ref[...]
    out_ref[...] += jnp.dot(x_ref[...], w_ref[...],
                            preferred_element_type=jnp.float32)


def _final_body(t_ref, bout_ref, bias_ref, out_ref):
    t = t_ref[...]
    bs = bout_ref[...] * jnp.float32(0.125)
    cols = []
    for c in range(OUT_DIM):
        s = jnp.sum(t[:, H * c:H * (c + 1)] * bs, axis=1, keepdims=True)
        cols.append(s + bias_ref[0:1, c:c + 1])
    cols.append(jnp.zeros((NB, 1), jnp.float32))
    out_ref[...] = jnp.concatenate(cols, axis=1)


SC_NBUF = 5
SC_OUTER = SC_NCHUNK // SC_NBUF


def _sc_gather_body(table_hbm, idx_hbm, out_hbm, *rest):
    idx_bufs = rest[:SC_NBUF]
    row_bufs = rest[SC_NBUF:2 * SC_NBUF]
    gsem, wsem = rest[2 * SC_NBUF], rest[2 * SC_NBUF + 1]
    wid = lax.axis_index("s") * SC_NC + lax.axis_index("c")

    def body(j, carry):
        def base(b):
            return pl.multiple_of(
                wid * E_PER_W + (j * SC_NBUF + b) * SC_CHUNK, 8)
        for b in range(SC_NBUF):
            pltpu.sync_copy(idx_hbm.at[pl.ds(base(b), SC_CHUNK)], idx_bufs[b])
        gh = [pltpu.async_copy(table_hbm.at[idx_bufs[b]], row_bufs[b], gsem)
              for b in range(SC_NBUF)]
        for b in range(SC_NBUF):
            gh[b].wait()
        wh = [pltpu.async_copy(
                  row_bufs[b], out_hbm.at[pl.ds(base(b), SC_CHUNK)], wsem)
              for b in range(SC_NBUF)]
        for b in range(SC_NBUF):
            wh[b].wait()
        return carry

    lax.fori_loop(0, SC_OUTER, body, 0)


def _sc_gather(table, idx_flat):
    fn = pl.kernel(
        _sc_gather_body,
        mesh=plsc.VectorSubcoreMesh(core_axis_name="c", subcore_axis_name="s",
                                    num_cores=SC_NC, num_subcores=SC_NS),
        out_type=jax.ShapeDtypeStruct((E_PAD, 128), jnp.float32),
        scratch_types=(
            [pltpu.VMEM((SC_CHUNK,), jnp.int32) for _ in range(SC_NBUF)]
            + [pltpu.VMEM((SC_CHUNK, 128), jnp.float32)
               for _ in range(SC_NBUF)]
            + [pltpu.SemaphoreType.DMA, pltpu.SemaphoreType.DMA]
        ),
    )
    return fn(table, idx_flat)


def _pad_rows(a, rows):
    return jnp.concatenate(
        [a, jnp.zeros((rows - a.shape[0],) + a.shape[1:], a.dtype)], axis=0)


def kernel(inp, out_grid_displacement, in_grid_displacement, initial_mesh,
           W_lift, b_lift, W_k1, b_k1, W_k2, b_k2, W_proj, b_proj,
           ln_g, ln_b, W_branch, b_branch, W_t1, b_t1, W_t2, b_t2, bias):
    f32 = jnp.float32
    im = _pad_rows(initial_mesh, NPAD)
    ind = _pad_rows(in_grid_displacement, NPAD)
    outd = _pad_rows(out_grid_displacement, NPAD)
    inp0 = _pad_rows(inp[0], NPAD)

    row = lambda v: v.reshape(1, -1)

    ing_p, table, tout = pl.pallas_call(
        _prep_body,
        grid=(NSTEPS,),
        in_specs=[
            pl.BlockSpec((NB, 2), lambda j: (j, 0)),
            pl.BlockSpec((NB, 2), lambda j: (j, 0)),
            pl.BlockSpec((NB, 2), lambda j: (j, 0)),
            pl.BlockSpec((NB, 3), lambda j: (j, 0)),
            pl.BlockSpec((19, H), lambda j: (0, 0)),
            pl.BlockSpec((1, H), lambda j: (0, 0)),
            pl.BlockSpec((18, H), lambda j: (0, 0)),
            pl.BlockSpec((1, H), lambda j: (0, 0)),
            pl.BlockSpec((H, H * OUT_DIM), lambda j: (0, 0)),
            pl.BlockSpec((1, H * OUT_DIM), lambda j: (0, 0)),
        ],
        out_specs=[
            pl.BlockSpec((NB, 2), lambda j: (j, 0)),
            pl.BlockSpec((NB, 128), lambda j: (j, 0)),
            pl.BlockSpec((NB, H * OUT_DIM), lambda j: (j, 0)),
        ],
        out_shape=[
            jax.ShapeDtypeStruct((NPAD, 2), f32),
            jax.ShapeDtypeStruct((NPAD, 128), f32),
            jax.ShapeDtypeStruct((NPAD, H * OUT_DIM), f32),
        ],
    )(im, ind, outd, inp0, W_lift, row(b_lift), W_t1, row(b_t1),
      W_t2, row(b_t2))

    qx = ing_p[:, 0:1]
    qy = ing_p[:, 1:2]
    px = jnp.full((1, CPAD), 1e9, f32).at[0, :N].set(ing_p[:N, 0])
    py = jnp.full((1, CPAD), 1e9, f32).at[0, :N].set(ing_p[:N, 1])
    pxr = px.reshape(8, NG).T
    pyr = py.reshape(8, NG).T
    nbrs16 = pl.pallas_call(
        _knn_body,
        grid=(NPAD // QB,),
        in_specs=[
            pl.BlockSpec((QB, 1), lambda j: (j, 0)),
            pl.BlockSpec((QB, 1), lambda j: (j, 0)),
            pl.BlockSpec((1, CPAD), lambda j: (0, 0)),
            pl.BlockSpec((1, CPAD), lambda j: (0, 0)),
            pl.BlockSpec((NG, 8), lambda j: (0, 0)),
            pl.BlockSpec((NG, 8), lambda j: (0, 0)),
        ],
        out_specs=pl.BlockSpec((QB, 16), lambda j: (j, 0)),
        out_shape=jax.ShapeDtypeStruct((NPAD, 16), jnp.int32),
    )(qx, qy, px, py, pxr, pyr)

    idx_flat = jnp.zeros((K, NPAD), jnp.int32)
    idx_flat = idx_flat.at[:, :N].set(nbrs16[:N, :K].T).reshape(E_PAD)

    g = _sc_gather(table, idx_flat)
    g3 = g.reshape(K, NPAD, 128)

    bnorm = pl.pallas_call(
        _edge_body,
        grid=(NSTEPS,),
        in_specs=[
            pl.BlockSpec((K, NB, 128), lambda j: (0, j, 0)),
            pl.BlockSpec((NB, 2), lambda j: (j, 0)),
            pl.BlockSpec((4, H), lambda j: (0, 0)),
            pl.BlockSpec((1, H), lambda j: (0, 0)),
            pl.BlockSpec((H, H), lambda j: (0, 0)),
            pl.BlockSpec((1, H), lambda j: (0, 0)),
            pl.BlockSpec((H, H), lambda j: (0, 0)),
            pl.BlockSpec((1, H), lambda j: (0, 0)),
            pl.BlockSpec((1, H), lambda j: (0, 0)),
            pl.BlockSpec((1, H), lambda j: (0, 0)),
        ],
        out_specs=pl.BlockSpec((NB, H), lambda j: (j, 0)),
        out_shape=jax.ShapeDtypeStruct((NPAD, H), f32),
    )(g3, ing_p, W_k1, row(b_k1), W_k2, row(b_k2), W_proj, row(b_proj),
      row(ln_g), row(ln_b))

    x = bnorm[:N].reshape(1, N * H)
    RB = 12800
    bout = pl.pallas_call(
        _branch_body,
        grid=(N * H // RB,),
        in_specs=[
            pl.BlockSpec((1, RB), lambda j: (0, j)),
            pl.BlockSpec((RB, H), lambda j: (j, 0)),
            pl.BlockSpec((1, H), lambda j: (0, 0)),
        ],
        out_specs=pl.BlockSpec((1, H), lambda j: (0, 0)),
        out_shape=jax.ShapeDtypeStruct((1, H), f32),
    )(x, W_branch, row(b_branch))

    bias128 = jnp.zeros((1, 128), f32).at[0, :OUT_DIM].set(bias)
    out_p = pl.pallas_call(
        _final_body,
        grid=(NSTEPS,),
        in_specs=[
            pl.BlockSpec((NB, H * OUT_DIM), lambda j: (j, 0)),
            pl.BlockSpec((1, H), lambda j: (0, 0)),
            pl.BlockSpec((1, 128), lambda j: (0, 0)),
        ],
        out_specs=pl.BlockSpec((NB, 4), lambda j: (j, 0)),
        out_shape=jax.ShapeDtypeStruct((NPAD, 4), f32),
    )(tout, bout, bias128)

    return out_p[:N, :OUT_DIM][None]
